# baseline (speedup 1.0000x reference)
import functools

import jax
import jax.numpy as jnp
from jax.experimental import pallas as pl

N = 50000
G = 50
NPG = 1000
NP = 1024
K = 16
NUM_CLASSES = 40
PAD_COORD = 1.0e4
BIG = 1.0e30


def _fused_graph_kernel(pos_ref, w1_ref, b1_ref, w2_ref, b2_ref,
                        w3_ref, b3_ref, w4_ref, b4_ref, wc_ref, bc_ref,
                        out_ref):
    f32 = jnp.float32
    p = pos_ref[0]
    pad = jnp.full((NP - NPG, 2), PAD_COORD, dtype=f32)
    pz = jnp.concatenate([p, pad], axis=0)

    px = pz[:, 0:1]
    py = pz[:, 1:2]
    dx = px - px.T
    dy = py - py.T
    d = dx * dx + dy * dy

    col = jax.lax.broadcasted_iota(jnp.int32, (NP, NP), 1)
    d = jnp.where(col >= NPG, BIG, d)

    w1 = w1_ref[...]
    u1 = jnp.dot(pz, w1[0:2] + w1[2:4], preferred_element_type=f32) + b1_ref[...]
    v1 = -jnp.dot(pz, w1[2:4], preferred_element_type=f32)
    w2 = w2_ref[...]
    b2 = b2_ref[...]
    bf16 = jnp.bfloat16
    u1_hi = u1.astype(bf16)
    u1_lo = (u1 - u1_hi.astype(f32)).astype(bf16)

    idxs = []
    m1 = jnp.full((NP, 32), -BIG, dtype=f32)
    for _ in range(K):
        rowmin = jnp.min(d, axis=1, keepdims=True)
        is_min = d == rowmin
        amin = jnp.min(jnp.where(is_min, col, 2 * NP), axis=1, keepdims=True)
        idxs.append(amin)
        d = jnp.where(col == amin, BIG, d)
        onehot = (col == amin).astype(bf16)
        g1 = (jnp.dot(onehot, u1_hi, preferred_element_type=f32)
              + jnp.dot(onehot, u1_lo, preferred_element_type=f32))
        z = jax.nn.relu(g1 + v1)
        msg = jnp.dot(z, w2, preferred_element_type=f32) + b2
        m1 = jnp.maximum(m1, msg)

    h1 = jax.nn.relu(m1)

    w3 = w3_ref[...]
    u2 = (jnp.dot(h1, w3[0:32], preferred_element_type=f32)
          + jnp.dot(pz, w3[32:34], preferred_element_type=f32) + b3_ref[...])
    v2 = -jnp.dot(pz, w3[32:34], preferred_element_type=f32)
    w4 = w4_ref[...]
    b4 = b4_ref[...]
    u2_hi = u2.astype(bf16)
    u2_lo = (u2 - u2_hi.astype(f32)).astype(bf16)

    m2 = jnp.full((NP, 32), -BIG, dtype=f32)
    for k in range(K):
        onehot = (col == idxs[k]).astype(bf16)
        g2 = (jnp.dot(onehot, u2_hi, preferred_element_type=f32)
              + jnp.dot(onehot, u2_lo, preferred_element_type=f32))
        z = jax.nn.relu(g2 + v2)
        msg = jnp.dot(z, w4, preferred_element_type=f32) + b4
        m2 = jnp.maximum(m2, msg)

    h2 = jax.nn.relu(m2)

    row = jax.lax.broadcasted_iota(jnp.int32, (NP, 32), 0)
    h2 = jnp.where(row < NPG, h2, -BIG)
    gvec = jnp.max(h2, axis=0, keepdims=True)
    logits = jnp.dot(gvec, wc_ref[...], preferred_element_type=f32) + bc_ref[...]
    out = jnp.pad(logits, ((0, 7), (0, 128 - NUM_CLASSES)))
    out_ref[0] = out


@functools.partial(jax.jit, static_argnames=("interpret",))
def _run(pos, W1, b1, W2, b2, W3, b3, W4, b4, Wc, bc, interpret=False):
    pos3 = pos.reshape(G, NPG, 2)
    full = lambda shape: pl.BlockSpec(shape, lambda g: (0,) * len(shape))
    out = pl.pallas_call(
        _fused_graph_kernel,
        grid=(G,),
        in_specs=[
            pl.BlockSpec((1, NPG, 2), lambda g: (g, 0, 0)),
            full((4, 32)), full((32,)), full((32, 32)), full((32,)),
            full((34, 32)), full((32,)), full((32, 32)), full((32,)),
            full((32, NUM_CLASSES)), full((NUM_CLASSES,)),
        ],
        out_specs=pl.BlockSpec((1, 8, 128), lambda g: (g, 0, 0)),
        out_shape=jax.ShapeDtypeStruct((G, 8, 128), jnp.float32),
        interpret=interpret,
    )(pos3, W1, b1, W2, b2, W3, b3, W4, b4, Wc, bc)
    return out[:, 0, :NUM_CLASSES]


def kernel(pos, batch, W1, b1, W2, b2, W3, b3, W4, b4, Wc, bc):
    del batch
    return _run(pos, W1, b1, W2, b2, W3, b3, W4, b4, Wc, bc)


# speedup vs baseline: 2.1072x; 2.1072x over previous
import functools

import jax
import jax.numpy as jnp
from jax.experimental import pallas as pl

N = 50000
G = 50
NPG = 1000
NP = 1024
K = 16
NUM_CLASSES = 40
PAD_COORD = 1.0e4
BIG = 1.0e30


def _fused_graph_kernel(posc_ref, posr_ref, w1t_ref, b1_ref, w2t_ref, b2_ref,
                        w3t_ref, b3_ref, w4t_ref, b4_ref, wc_ref, bc_ref,
                        out_ref):
    f32 = jnp.float32
    pc = posc_ref[0]
    pr = posr_ref[0]

    dx = pc[:, 0:1] - pr[0:1, :]
    dy = pc[:, 1:2] - pr[1:2, :]
    d = dx * dx + dy * dy

    rowi = jax.lax.broadcasted_iota(jnp.int32, (NP, NP), 0)
    d = jnp.where(rowi >= NPG, BIG, d)

    w1t = w1t_ref[...]
    u1 = jnp.dot(w1t[:, 0:2] + w1t[:, 2:4], pr, preferred_element_type=f32) \
        + b1_ref[...].reshape(32, 1)
    v1 = -jnp.dot(w1t[:, 2:4], pr, preferred_element_type=f32)
    w2t = w2t_ref[...]
    b2 = b2_ref[...].reshape(32, 1)

    idxs = []
    m1 = jnp.full((32, NP), -BIG, dtype=f32)
    for _ in range(K):
        cmin = jnp.min(d, axis=0, keepdims=True)
        is_min = d == cmin
        amin = jnp.min(jnp.where(is_min, rowi, 2 * NP), axis=0, keepdims=True)
        idxs.append(amin)
        sel = rowi == amin
        d = jnp.where(sel, BIG, d)
        g1 = jnp.dot(u1, sel.astype(f32), preferred_element_type=f32)
        z = jax.nn.relu(g1 + v1)
        msg = jnp.dot(w2t, z, preferred_element_type=f32) + b2
        m1 = jnp.maximum(m1, msg)

    h1 = jax.nn.relu(m1)

    w3t = w3t_ref[...]
    u2 = (jnp.dot(w3t[:, 0:32], h1, preferred_element_type=f32)
          + jnp.dot(w3t[:, 32:34], pr, preferred_element_type=f32)
          + b3_ref[...].reshape(32, 1))
    v2 = -jnp.dot(w3t[:, 32:34], pr, preferred_element_type=f32)
    w4t = w4t_ref[...]
    b4 = b4_ref[...].reshape(32, 1)

    m2 = jnp.full((32, NP), -BIG, dtype=f32)
    for k in range(K):
        sel = (rowi == idxs[k]).astype(f32)
        g2 = jnp.dot(u2, sel, preferred_element_type=f32)
        z = jax.nn.relu(g2 + v2)
        msg = jnp.dot(w4t, z, preferred_element_type=f32) + b4
        m2 = jnp.maximum(m2, msg)

    h2 = jax.nn.relu(m2)

    coli = jax.lax.broadcasted_iota(jnp.int32, (32, NP), 1)
    h2 = jnp.where(coli < NPG, h2, -BIG)
    gvec = jnp.max(h2, axis=1).reshape(1, 32)
    logits = jnp.dot(gvec, wc_ref[...], preferred_element_type=f32) + bc_ref[...]
    out = jnp.pad(logits, ((0, 7), (0, 128 - NUM_CLASSES)))
    out_ref[0] = out


@functools.partial(jax.jit, static_argnames=("interpret",))
def _run(pos, W1, b1, W2, b2, W3, b3, W4, b4, Wc, bc, interpret=False):
    pos3 = pos.reshape(G, NPG, 2)
    padc = jnp.full((G, NP - NPG, 2), PAD_COORD, dtype=pos.dtype)
    posc = jnp.concatenate([pos3, padc], axis=1)
    posr = posc.transpose(0, 2, 1)
    full = lambda shape: pl.BlockSpec(shape, lambda g: (0,) * len(shape))
    out = pl.pallas_call(
        _fused_graph_kernel,
        grid=(G,),
        in_specs=[
            pl.BlockSpec((1, NP, 2), lambda g: (g, 0, 0)),
            pl.BlockSpec((1, 2, NP), lambda g: (g, 0, 0)),
            full((32, 4)), full((32,)), full((32, 32)), full((32,)),
            full((32, 34)), full((32,)), full((32, 32)), full((32,)),
            full((32, NUM_CLASSES)), full((NUM_CLASSES,)),
        ],
        out_specs=pl.BlockSpec((1, 8, 128), lambda g: (g, 0, 0)),
        out_shape=jax.ShapeDtypeStruct((G, 8, 128), jnp.float32),
        interpret=interpret,
    )(posc, posr, W1.T, b1, W2.T, b2, W3.T, b3, W4.T, b4, Wc, bc)
    return out[:, 0, :NUM_CLASSES]


def kernel(pos, batch, W1, b1, W2, b2, W3, b3, W4, b4, Wc, bc):
    del batch
    return _run(pos, W1, b1, W2, b2, W3, b3, W4, b4, Wc, bc)
